# baseline (speedup 1.0000x reference)
import functools

import jax
import jax.numpy as jnp
from jax import lax
from jax.experimental import pallas as pl
from jax.experimental.pallas import tpu as pltpu
from jax.experimental.pallas import tpu_sc as plsc

B = 1024
K = 100000
D = 768
TOPK = 5
NS = 8
CK = 2048
EPS = 1e-8
NEG_MASKED = -1.0e30
NEG_TAKEN = -2.0e30
NEG_INIT = -3.0e30
IBIG = 2 ** 30


LN = 128
NCH = CK // LN


def _topk_body(q_ref, tsk_ref, tr_ref, spk_ref, out_ref, qn_ref, cv_ref, ci_ref):
    j = pl.program_id(0)

    @pl.when(j == 0)
    def _init():
        qv = q_ref[...]
        qnorm = jnp.sqrt(jnp.sum(qv * qv, axis=1, keepdims=True))
        qn_ref[...] = (qv / jnp.maximum(qnorm, EPS)).astype(jnp.bfloat16)
        cv_ref[...] = jnp.full((B, TOPK * LN), NEG_INIT, jnp.float32)
        ci_ref[...] = jnp.full((B, TOPK * LN), IBIG, jnp.int32)

    c = tr_ref[...]
    cnorm2 = jnp.sum(c * c, axis=1)
    rinv = 1.0 / jnp.maximum(jnp.sqrt(cnorm2), EPS)
    cn = (c * rinv[:, None]).astype(jnp.bfloat16)
    sims = lax.dot_general(
        qn_ref[...], cn, (((1,), (1,)), ((), ())),
        preferred_element_type=jnp.float32)

    V = [cv_ref[:, t * LN:(t + 1) * LN] for t in range(TOPK)]
    I = [ci_ref[:, t * LN:(t + 1) * LN] for t in range(TOPK)]
    tsk = tsk_ref[...]
    lane = lax.broadcasted_iota(jnp.int32, (1, LN), 1)
    for k in range(NCH):
        base = j * CK + k * LN
        gcol = lane + base
        ok = (tsk == spk_ref[:, k * LN:(k + 1) * LN]) & (gcol < K)
        x = jnp.where(ok, sims[:, k * LN:(k + 1) * LN], NEG_MASKED)
        b = [x > V[t] for t in range(TOPK)]
        nV, nI = [], []
        for t in range(TOPK):
            if t == 0:
                pv, pi = x, gcol
            else:
                pv = jnp.where(b[t - 1], V[t - 1], x)
                pi = jnp.where(b[t - 1], I[t - 1], gcol)
            nV.append(jnp.where(b[t], pv, V[t]))
            nI.append(jnp.where(b[t], pi, I[t]))
        V, I = nV, nI
    for t in range(TOPK):
        cv_ref[:, t * LN:(t + 1) * LN] = V[t]
        ci_ref[:, t * LN:(t + 1) * LN] = I[t]

    @pl.when(j == pl.num_programs(0) - 1)
    def _emit():
        cv = cv_ref[...]
        ci = ci_ref[...]
        outs = []
        for _ in range(TOPK):
            m = jnp.max(cv, axis=1, keepdims=True)
            hit = cv == m
            il = jnp.min(jnp.where(hit, ci, IBIG), axis=1, keepdims=True)
            outs.append(il)
            cv = jnp.where(hit & (ci == il), NEG_TAKEN, cv)
        outs += [outs[0]] * (NS - TOPK)
        out_ref[...] = jnp.concatenate(outs, axis=1)


def _topk_call(q, tsk2, train, spk2):
    nsteps = (K + CK - 1) // CK
    return pl.pallas_call(
        _topk_body,
        grid=(nsteps,),
        in_specs=[
            pl.BlockSpec((B, D), lambda j: (0, 0)),
            pl.BlockSpec((B, 1), lambda j: (0, 0)),
            pl.BlockSpec((CK, D), lambda j: (j, 0)),
            pl.BlockSpec((1, CK), lambda j: (0, j)),
        ],
        out_specs=pl.BlockSpec((B, NS), lambda j: (0, 0)),
        out_shape=jax.ShapeDtypeStruct((B, NS), jnp.int32),
        scratch_shapes=[
            pltpu.VMEM((B, D), jnp.bfloat16),
            pltpu.VMEM((B, TOPK * LN), jnp.float32),
            pltpu.VMEM((B, TOPK * LN), jnp.int32),
        ],
        compiler_params=pltpu.CompilerParams(
            dimension_semantics=("arbitrary",)),
    )(q, tsk2, train, spk2)


_SC_NC = 2
_SC_NSUB = 16
_NW = _SC_NC * _SC_NSUB
_QPW = B // _NW
_RQ = 8
_NROUND = _QPW // _RQ
_ROWS = _RQ * TOPK
_NSL = D // 16


def _gather_mean_body(idx_hbm, tr_hbm, out_hbm, idx_v, rows_v, out_v, sem):
    wid = lax.axis_index("s") * _SC_NC + lax.axis_index("c")
    for r in range(_NROUND):
        base_q = wid * _QPW + r * _RQ
        pltpu.sync_copy(idx_hbm.at[pl.ds(base_q * TOPK, _ROWS)], idx_v)
        pltpu.async_copy(tr_hbm.at[idx_v], rows_v, sem).wait()
        for q in range(_RQ):
            def slice_body(t, carry, q=q):
                sl = pl.ds(t * 16, 16)
                acc = rows_v[TOPK * q, sl]
                for rr in range(1, TOPK):
                    acc = acc + rows_v[TOPK * q + rr, sl]
                out_v[q, sl] = acc * (1.0 / TOPK)
                return carry
            lax.fori_loop(0, _NSL, slice_body, 0)
        pltpu.sync_copy(out_v, out_hbm.at[pl.ds(base_q, _RQ)])


@functools.cache
def _gather_mean_kernel():
    return pl.kernel(
        _gather_mean_body,
        out_type=jax.ShapeDtypeStruct((B, D), jnp.float32),
        mesh=plsc.VectorSubcoreMesh(core_axis_name="c", subcore_axis_name="s",
                                    num_cores=_SC_NC, num_subcores=_SC_NSUB),
        scratch_types=[
            pltpu.VMEM((_ROWS,), jnp.int32),
            pltpu.VMEM((_ROWS, D), jnp.float32),
            pltpu.VMEM((_RQ, D), jnp.float32),
            pltpu.SemaphoreType.DMA,
        ],
    )


def _mlp_body(q_ref, r_ref, w1_ref, b1_ref, w2_ref, b2_ref, out_ref):
    w1 = w1_ref[...]
    h = lax.dot_general(
        q_ref[...], w1[:D, :], (((1,), (0,)), ((), ())),
        precision=lax.Precision.HIGHEST,
        preferred_element_type=jnp.float32)
    h = h + lax.dot_general(
        r_ref[...], w1[D:, :], (((1,), (0,)), ((), ())),
        precision=lax.Precision.HIGHEST,
        preferred_element_type=jnp.float32)
    h = h + b1_ref[...]
    h = h * (1.0 / (1.0 + jnp.exp(-h)))
    out = lax.dot_general(
        h, w2_ref[...], (((1,), (0,)), ((), ())),
        precision=lax.Precision.HIGHEST,
        preferred_element_type=jnp.float32)
    out_ref[...] = out + b2_ref[...]


def _mlp_call(q, retrieved, w1, b1, w2, b2):
    return pl.pallas_call(
        _mlp_body,
        out_shape=jax.ShapeDtypeStruct((B, D), jnp.float32),
    )(q, retrieved, w1, b1.reshape(1, D), w2, b2.reshape(1, D))


def kernel(content_features, target_speaker_id, training_features,
           speaker_ids, W1, b1, W2, b2):
    tsk2 = target_speaker_id.reshape(B, 1)
    spk2 = speaker_ids.reshape(1, K)
    top_idx = _topk_call(content_features, tsk2, training_features, spk2)
    idx_flat = top_idx[:, :TOPK].reshape(B * TOPK)
    retrieved = _gather_mean_kernel()(idx_flat, training_features)
    return _mlp_call(content_features, retrieved, W1, b1, W2, b2)


# speedup vs baseline: 1.0298x; 1.0298x over previous
import functools

import jax
import jax.numpy as jnp
from jax import lax
from jax.experimental import pallas as pl
from jax.experimental.pallas import tpu as pltpu
from jax.experimental.pallas import tpu_sc as plsc

B = 1024
K = 100000
D = 768
TOPK = 5
NS = 8
CK = 1536
EPS = 1e-8
NEG_MASKED = -1.0e30
NEG_TAKEN = -2.0e30
NEG_INIT = -3.0e30
IBIG = 2 ** 30


LN = 128
NCH = CK // LN


def _produce(tr_ref, spk_ref, qn, tsk, bound, dst_ref):
    c = tr_ref[...]
    cnorm2 = jnp.sum(c * c, axis=1)
    rinv = 1.0 / jnp.maximum(jnp.sqrt(cnorm2), EPS)
    cn = (c * rinv[:, None]).astype(jnp.bfloat16)
    dots = lax.dot_general(
        qn, cn, (((1,), (1,)), ((), ())),
        preferred_element_type=jnp.float32)
    inb = lax.broadcasted_iota(jnp.int32, (1, CK), 1) < bound
    ok = (tsk == spk_ref[...]) & inb
    dst_ref[...] = jnp.where(ok, dots, NEG_MASKED)


def _consume(src_ref, base, lane, V, I):
    for k in range(NCH):
        gcol = lane + (base + k * LN)
        x = src_ref[:, k * LN:(k + 1) * LN]
        b = [x > V[t] for t in range(TOPK)]
        nV, nI = [], []
        for t in range(TOPK):
            if t == 0:
                pv, pi = x, gcol
            else:
                pv = jnp.where(b[t - 1], V[t - 1], x)
                pi = jnp.where(b[t - 1], I[t - 1], gcol)
            nV.append(jnp.where(b[t], pv, V[t]))
            nI.append(jnp.where(b[t], pi, I[t]))
        V, I = nV, nI
    return V, I


def _topk_body(q_ref, tsk_ref, tr0_ref, spk0_ref, tr1_ref, spk1_ref,
               out_ref, qn_ref, s0_ref, s1_ref, cv_ref, ci_ref):
    i = pl.program_id(0)

    @pl.when(i == 0)
    def _init():
        qv = q_ref[...]
        qnorm = jnp.sqrt(jnp.sum(qv * qv, axis=1, keepdims=True))
        qn_ref[...] = (qv / jnp.maximum(qnorm, EPS)).astype(jnp.bfloat16)
        cv_ref[...] = jnp.full((B, TOPK * LN), NEG_INIT, jnp.float32)
        ci_ref[...] = jnp.full((B, TOPK * LN), IBIG, jnp.int32)
        s1_ref[...] = jnp.full((B, CK), NEG_INIT, jnp.float32)

    qn = qn_ref[...]
    tsk = tsk_ref[...]
    lane = lax.broadcasted_iota(jnp.int32, (1, LN), 1)
    V = [cv_ref[:, t * LN:(t + 1) * LN] for t in range(TOPK)]
    I = [ci_ref[:, t * LN:(t + 1) * LN] for t in range(TOPK)]

    _produce(tr0_ref, spk0_ref, qn, tsk, K - (2 * i) * CK, s0_ref)
    V, I = _consume(s1_ref, (2 * i - 1) * CK, lane, V, I)
    _produce(tr1_ref, spk1_ref, qn, tsk, K - (2 * i + 1) * CK, s1_ref)
    V, I = _consume(s0_ref, (2 * i) * CK, lane, V, I)

    for t in range(TOPK):
        cv_ref[:, t * LN:(t + 1) * LN] = V[t]
        ci_ref[:, t * LN:(t + 1) * LN] = I[t]

    @pl.when(i == pl.num_programs(0) - 1)
    def _emit():
        cv = cv_ref[...]
        ci = ci_ref[...]
        outs = []
        for _ in range(TOPK):
            m = jnp.max(cv, axis=1, keepdims=True)
            hit = cv == m
            il = jnp.min(jnp.where(hit, ci, IBIG), axis=1, keepdims=True)
            outs.append(il)
            cv = jnp.where(hit & (ci == il), NEG_TAKEN, cv)
        outs += [outs[0]] * (NS - TOPK)
        out_ref[...] = jnp.concatenate(outs, axis=1)


def _topk_call(q, tsk2, train, spk2):
    nchunk = (K + CK - 1) // CK
    ng = nchunk // 2 + 1
    last = nchunk - 1
    return pl.pallas_call(
        _topk_body,
        grid=(ng,),
        in_specs=[
            pl.BlockSpec((B, D), lambda i: (0, 0)),
            pl.BlockSpec((B, 1), lambda i: (0, 0)),
            pl.BlockSpec((CK, D), lambda i: (jnp.minimum(2 * i, last), 0)),
            pl.BlockSpec((1, CK), lambda i: (0, jnp.minimum(2 * i, last))),
            pl.BlockSpec((CK, D), lambda i: (jnp.minimum(2 * i + 1, last), 0)),
            pl.BlockSpec((1, CK), lambda i: (0, jnp.minimum(2 * i + 1, last))),
        ],
        out_specs=pl.BlockSpec((B, NS), lambda i: (0, 0)),
        out_shape=jax.ShapeDtypeStruct((B, NS), jnp.int32),
        scratch_shapes=[
            pltpu.VMEM((B, D), jnp.bfloat16),
            pltpu.VMEM((B, CK), jnp.float32),
            pltpu.VMEM((B, CK), jnp.float32),
            pltpu.VMEM((B, TOPK * LN), jnp.float32),
            pltpu.VMEM((B, TOPK * LN), jnp.int32),
        ],
        compiler_params=pltpu.CompilerParams(
            dimension_semantics=("arbitrary",)),
    )(q, tsk2, train, spk2, train, spk2)


_SC_NC = 2
_SC_NSUB = 16
_NW = _SC_NC * _SC_NSUB
_QPW = B // _NW
_RQ = 8
_NROUND = _QPW // _RQ
_ROWS = _RQ * TOPK
_NSL = D // 16


def _gather_mean_body(idx_hbm, tr_hbm, out_hbm, idx_v, rows_v, out_v, sem):
    wid = lax.axis_index("s") * _SC_NC + lax.axis_index("c")
    for r in range(_NROUND):
        base_q = wid * _QPW + r * _RQ
        pltpu.sync_copy(idx_hbm.at[pl.ds(base_q * TOPK, _ROWS)], idx_v)
        pltpu.async_copy(tr_hbm.at[idx_v], rows_v, sem).wait()
        for q in range(_RQ):
            def slice_body(t, carry, q=q):
                sl = pl.ds(t * 16, 16)
                acc = rows_v[TOPK * q, sl]
                for rr in range(1, TOPK):
                    acc = acc + rows_v[TOPK * q + rr, sl]
                out_v[q, sl] = acc * (1.0 / TOPK)
                return carry
            lax.fori_loop(0, _NSL, slice_body, 0)
        pltpu.sync_copy(out_v, out_hbm.at[pl.ds(base_q, _RQ)])


@functools.cache
def _gather_mean_kernel():
    return pl.kernel(
        _gather_mean_body,
        out_type=jax.ShapeDtypeStruct((B, D), jnp.float32),
        mesh=plsc.VectorSubcoreMesh(core_axis_name="c", subcore_axis_name="s",
                                    num_cores=_SC_NC, num_subcores=_SC_NSUB),
        scratch_types=[
            pltpu.VMEM((_ROWS,), jnp.int32),
            pltpu.VMEM((_ROWS, D), jnp.float32),
            pltpu.VMEM((_RQ, D), jnp.float32),
            pltpu.SemaphoreType.DMA,
        ],
    )


def _mlp_body(q_ref, r_ref, w1_ref, b1_ref, w2_ref, b2_ref, out_ref):
    w1 = w1_ref[...]
    h = lax.dot_general(
        q_ref[...], w1[:D, :], (((1,), (0,)), ((), ())),
        precision=lax.Precision.HIGHEST,
        preferred_element_type=jnp.float32)
    h = h + lax.dot_general(
        r_ref[...], w1[D:, :], (((1,), (0,)), ((), ())),
        precision=lax.Precision.HIGHEST,
        preferred_element_type=jnp.float32)
    h = h + b1_ref[...]
    h = h * (1.0 / (1.0 + jnp.exp(-h)))
    out = lax.dot_general(
        h, w2_ref[...], (((1,), (0,)), ((), ())),
        precision=lax.Precision.HIGHEST,
        preferred_element_type=jnp.float32)
    out_ref[...] = out + b2_ref[...]


def _mlp_call(q, retrieved, w1, b1, w2, b2):
    return pl.pallas_call(
        _mlp_body,
        out_shape=jax.ShapeDtypeStruct((B, D), jnp.float32),
    )(q, retrieved, w1, b1.reshape(1, D), w2, b2.reshape(1, D))


def kernel(content_features, target_speaker_id, training_features,
           speaker_ids, W1, b1, W2, b2):
    tsk2 = target_speaker_id.reshape(B, 1)
    spk2 = speaker_ids.reshape(1, K)
    top_idx = _topk_call(content_features, tsk2, training_features, spk2)
    idx_flat = top_idx[:, :TOPK].reshape(B * TOPK)
    retrieved = _gather_mean_kernel()(idx_flat, training_features)
    return _mlp_call(content_features, retrieved, W1, b1, W2, b2)


# speedup vs baseline: 1.0660x; 1.0351x over previous
import functools

import jax
import jax.numpy as jnp
from jax import lax
from jax.experimental import pallas as pl
from jax.experimental.pallas import tpu as pltpu
from jax.experimental.pallas import tpu_sc as plsc

B = 1024
K = 100000
D = 768
TOPK = 5
NS = 8
CK = 1536
EPS = 1e-8
NEG_MASKED = -1.0e30
NEG_TAKEN = -2.0e30
NEG_INIT = -3.0e30
IBIG = 2 ** 30


LN = 128
NCH = CK // LN


def _produce(tr_ref, spk_ref, qn, tsk, bound, dst_ref):
    c = tr_ref[...]
    cnorm2 = jnp.sum(c * c, axis=1)
    rinv = 1.0 / jnp.maximum(jnp.sqrt(cnorm2), EPS)
    cn = (c * rinv[:, None]).astype(jnp.bfloat16)
    dots = lax.dot_general(
        qn, cn, (((1,), (1,)), ((), ())),
        preferred_element_type=jnp.float32)
    inb = lax.broadcasted_iota(jnp.int32, (1, CK), 1) < bound
    spk_adj = jnp.where(inb, spk_ref[...], -1)
    dst_ref[...] = jnp.where(tsk == spk_adj, dots, NEG_MASKED)


def _consume(src_ref, base, lane, V, I):
    for k in range(NCH):
        gcol = lane + (base + k * LN)
        x = src_ref[:, k * LN:(k + 1) * LN]
        b = [x > V[t] for t in range(TOPK)]
        nV, nI = [], []
        for t in range(TOPK):
            if t == 0:
                pv, pi = x, gcol
            else:
                pv = jnp.where(b[t - 1], V[t - 1], x)
                pi = jnp.where(b[t - 1], I[t - 1], gcol)
            nV.append(jnp.where(b[t], pv, V[t]))
            nI.append(jnp.where(b[t], pi, I[t]))
        V, I = nV, nI
    return V, I


def _topk_body(q_ref, tsk_ref, tr0_ref, spk0_ref, tr1_ref, spk1_ref,
               out_ref, qn_ref, s0_ref, s1_ref, cv_ref, ci_ref):
    i = pl.program_id(0)

    @pl.when(i == 0)
    def _init():
        qv = q_ref[...]
        qnorm = jnp.sqrt(jnp.sum(qv * qv, axis=1, keepdims=True))
        qn_ref[...] = (qv / jnp.maximum(qnorm, EPS)).astype(jnp.bfloat16)
        cv_ref[...] = jnp.full((B, TOPK * LN), NEG_INIT, jnp.float32)
        ci_ref[...] = jnp.full((B, TOPK * LN), IBIG, jnp.int32)
        s1_ref[...] = jnp.full((B, CK), NEG_INIT, jnp.float32)

    qn = qn_ref[...]
    tsk = tsk_ref[...]
    lane = lax.broadcasted_iota(jnp.int32, (1, LN), 1)
    V = [cv_ref[:, t * LN:(t + 1) * LN] for t in range(TOPK)]
    I = [ci_ref[:, t * LN:(t + 1) * LN] for t in range(TOPK)]

    _produce(tr0_ref, spk0_ref, qn, tsk, K - (2 * i) * CK, s0_ref)
    V, I = _consume(s1_ref, (2 * i - 1) * CK, lane, V, I)
    _produce(tr1_ref, spk1_ref, qn, tsk, K - (2 * i + 1) * CK, s1_ref)
    V, I = _consume(s0_ref, (2 * i) * CK, lane, V, I)

    for t in range(TOPK):
        cv_ref[:, t * LN:(t + 1) * LN] = V[t]
        ci_ref[:, t * LN:(t + 1) * LN] = I[t]

    @pl.when(i == pl.num_programs(0) - 1)
    def _emit():
        cv = cv_ref[...]
        ci = ci_ref[...]
        outs = []
        for _ in range(TOPK):
            m = jnp.max(cv, axis=1, keepdims=True)
            hit = cv == m
            il = jnp.min(jnp.where(hit, ci, IBIG), axis=1, keepdims=True)
            outs.append(il)
            cv = jnp.where(hit & (ci == il), NEG_TAKEN, cv)
        outs += [outs[0]] * (NS - TOPK)
        out_ref[...] = jnp.concatenate(outs, axis=1)


def _topk_call(q, tsk2, train, spk2):
    nchunk = (K + CK - 1) // CK
    ng = nchunk // 2 + 1
    last = nchunk - 1
    return pl.pallas_call(
        _topk_body,
        grid=(ng,),
        in_specs=[
            pl.BlockSpec((B, D), lambda i: (0, 0)),
            pl.BlockSpec((B, 1), lambda i: (0, 0)),
            pl.BlockSpec((CK, D), lambda i: (jnp.minimum(2 * i, last), 0)),
            pl.BlockSpec((1, CK), lambda i: (0, jnp.minimum(2 * i, last))),
            pl.BlockSpec((CK, D), lambda i: (jnp.minimum(2 * i + 1, last), 0)),
            pl.BlockSpec((1, CK), lambda i: (0, jnp.minimum(2 * i + 1, last))),
        ],
        out_specs=pl.BlockSpec((B, NS), lambda i: (0, 0)),
        out_shape=jax.ShapeDtypeStruct((B, NS), jnp.int32),
        scratch_shapes=[
            pltpu.VMEM((B, D), jnp.bfloat16),
            pltpu.VMEM((B, CK), jnp.float32),
            pltpu.VMEM((B, CK), jnp.float32),
            pltpu.VMEM((B, TOPK * LN), jnp.float32),
            pltpu.VMEM((B, TOPK * LN), jnp.int32),
        ],
        compiler_params=pltpu.CompilerParams(
            dimension_semantics=("arbitrary",)),
    )(q, tsk2, train, spk2, train, spk2)


_SC_NC = 2
_SC_NSUB = 16
_NW = _SC_NC * _SC_NSUB
_QPW = B // _NW
_RQ = 8
_NROUND = _QPW // _RQ
_ROWS = _RQ * TOPK
_NSL = D // 16


def _gather_mean_body(idx_hbm, tr_hbm, out_hbm, idx_v, rows_v, out_v, sem):
    wid = lax.axis_index("s") * _SC_NC + lax.axis_index("c")
    for r in range(_NROUND):
        base_q = wid * _QPW + r * _RQ
        pltpu.sync_copy(idx_hbm.at[pl.ds(base_q * TOPK, _ROWS)], idx_v)
        pltpu.async_copy(tr_hbm.at[idx_v], rows_v, sem).wait()
        for q in range(_RQ):
            def slice_body(t, carry, q=q):
                sl = pl.ds(t * 16, 16)
                acc = rows_v[TOPK * q, sl]
                for rr in range(1, TOPK):
                    acc = acc + rows_v[TOPK * q + rr, sl]
                out_v[q, sl] = acc * (1.0 / TOPK)
                return carry
            lax.fori_loop(0, _NSL, slice_body, 0)
        pltpu.sync_copy(out_v, out_hbm.at[pl.ds(base_q, _RQ)])


@functools.cache
def _gather_mean_kernel():
    return pl.kernel(
        _gather_mean_body,
        out_type=jax.ShapeDtypeStruct((B, D), jnp.float32),
        mesh=plsc.VectorSubcoreMesh(core_axis_name="c", subcore_axis_name="s",
                                    num_cores=_SC_NC, num_subcores=_SC_NSUB),
        scratch_types=[
            pltpu.VMEM((_ROWS,), jnp.int32),
            pltpu.VMEM((_ROWS, D), jnp.float32),
            pltpu.VMEM((_RQ, D), jnp.float32),
            pltpu.SemaphoreType.DMA,
        ],
    )


def _mlp_body(q_ref, r_ref, w1_ref, b1_ref, w2_ref, b2_ref, out_ref):
    w1 = w1_ref[...]
    h = lax.dot_general(
        q_ref[...], w1[:D, :], (((1,), (0,)), ((), ())),
        precision=lax.Precision.HIGHEST,
        preferred_element_type=jnp.float32)
    h = h + lax.dot_general(
        r_ref[...], w1[D:, :], (((1,), (0,)), ((), ())),
        precision=lax.Precision.HIGHEST,
        preferred_element_type=jnp.float32)
    h = h + b1_ref[...]
    h = h * (1.0 / (1.0 + jnp.exp(-h)))
    out = lax.dot_general(
        h, w2_ref[...], (((1,), (0,)), ((), ())),
        precision=lax.Precision.HIGHEST,
        preferred_element_type=jnp.float32)
    out_ref[...] = out + b2_ref[...]


def _mlp_call(q, retrieved, w1, b1, w2, b2):
    return pl.pallas_call(
        _mlp_body,
        out_shape=jax.ShapeDtypeStruct((B, D), jnp.float32),
    )(q, retrieved, w1, b1.reshape(1, D), w2, b2.reshape(1, D))


def kernel(content_features, target_speaker_id, training_features,
           speaker_ids, W1, b1, W2, b2):
    tsk2 = target_speaker_id.reshape(B, 1)
    spk2 = speaker_ids.reshape(1, K)
    top_idx = _topk_call(content_features, tsk2, training_features, spk2)
    idx_flat = top_idx[:, :TOPK].reshape(B * TOPK)
    retrieved = _gather_mean_kernel()(idx_flat, training_features)
    return _mlp_call(content_features, retrieved, W1, b1, W2, b2)


# speedup vs baseline: 1.1729x; 1.1003x over previous
import functools

import jax
import jax.numpy as jnp
from jax import lax
from jax.experimental import pallas as pl
from jax.experimental.pallas import tpu as pltpu
from jax.experimental.pallas import tpu_sc as plsc

B = 1024
K = 100000
D = 768
TOPK = 5
NS = 8
CK = 1280
EPS = 1e-8
NEG_MASKED = -1.0e30
NEG_TAKEN = -2.0e30
NEG_INIT = -3.0e30
IBIG = 2 ** 30


LN = 128
NCH = CK // LN


def _produce(tr_ref, spk_ref, qn, tsk, bound, dst_ref):
    c = tr_ref[...]
    cnorm2 = jnp.sum(c * c, axis=1)
    rinv = 1.0 / jnp.maximum(jnp.sqrt(cnorm2), EPS)
    cn = (c * rinv[:, None]).astype(jnp.bfloat16)
    dots = lax.dot_general(
        qn, cn, (((1,), (1,)), ((), ())),
        preferred_element_type=jnp.float32)
    inb = lax.broadcasted_iota(jnp.int32, (1, CK), 1) < bound
    spk_adj = jnp.where(inb, spk_ref[...], -1)
    dst_ref[...] = jnp.where(tsk == spk_adj, dots, NEG_MASKED)


HALF = NCH // 2


def _consume(src_ref, ch, lane, V, L, P):
    for k in range(HALF):
        x1 = src_ref[:, k * LN:(k + 1) * LN]
        x2 = src_ref[:, (k + HALF) * LN:(k + HALF + 1) * LN]
        b12 = x2 > x1
        hi = jnp.maximum(x1, x2)
        lo = jnp.minimum(x1, x2)
        pid2 = (lane + (ch * HALF + k) * LN) * 2 + b12.astype(jnp.int32)
        b = [hi > V[t] for t in range(TOPK)]
        nV, nL, nP = [], [], []
        for t in range(TOPK):
            if t == 0:
                pv, pl_, pp = hi, lo, pid2
            else:
                pv = jnp.where(b[t - 1], V[t - 1], hi)
                pl_ = jnp.where(b[t - 1], L[t - 1], lo)
                pp = jnp.where(b[t - 1], P[t - 1], pid2)
            nV.append(jnp.where(b[t], pv, V[t]))
            nL.append(jnp.where(b[t], pl_, L[t]))
            nP.append(jnp.where(b[t], pp, P[t]))
        V, L, P = nV, nL, nP
    return V, L, P


def _topk_body(q_ref, tsk_ref, tr0_ref, spk0_ref, tr1_ref, spk1_ref,
               out_ref, qn_ref, s0_ref, s1_ref, cv_ref, cl_ref, ci_ref):
    i = pl.program_id(0)

    @pl.when(i == 0)
    def _init():
        qv = q_ref[...]
        qnorm = jnp.sqrt(jnp.sum(qv * qv, axis=1, keepdims=True))
        qn_ref[...] = (qv / jnp.maximum(qnorm, EPS)).astype(jnp.bfloat16)
        cv_ref[...] = jnp.full((B, TOPK * LN), NEG_INIT, jnp.float32)
        cl_ref[...] = jnp.full((B, TOPK * LN), NEG_INIT, jnp.float32)
        ci_ref[...] = jnp.zeros((B, TOPK * LN), jnp.int32)
        s1_ref[...] = jnp.full((B, CK), NEG_INIT, jnp.float32)

    qn = qn_ref[...]
    tsk = tsk_ref[...]
    lane = lax.broadcasted_iota(jnp.int32, (1, LN), 1)
    V = [cv_ref[:, t * LN:(t + 1) * LN] for t in range(TOPK)]
    L = [cl_ref[:, t * LN:(t + 1) * LN] for t in range(TOPK)]
    P = [ci_ref[:, t * LN:(t + 1) * LN] for t in range(TOPK)]

    _produce(tr0_ref, spk0_ref, qn, tsk, K - (2 * i) * CK, s0_ref)
    V, L, P = _consume(s1_ref, 2 * i - 1, lane, V, L, P)
    _produce(tr1_ref, spk1_ref, qn, tsk, K - (2 * i + 1) * CK, s1_ref)
    V, L, P = _consume(s0_ref, 2 * i, lane, V, L, P)

    for t in range(TOPK):
        cv_ref[:, t * LN:(t + 1) * LN] = V[t]
        cl_ref[:, t * LN:(t + 1) * LN] = L[t]
        ci_ref[:, t * LN:(t + 1) * LN] = P[t]

    @pl.when(i == pl.num_programs(0) - 1)
    def _emit():
        pid2 = ci_ref[...]
        flag = pid2 & 1
        pid = pid2 >> 1
        g = pid >> 7
        ln_ = pid & (LN - 1)
        c1 = (g // HALF) * CK + (g % HALF) * LN + ln_
        hic = c1 + flag * (HALF * LN)
        loc = c1 + (1 - flag) * (HALF * LN)
        cv = jnp.concatenate([cv_ref[...], cl_ref[...]], axis=1)
        ci = jnp.concatenate([hic, loc], axis=1)
        outs = []
        for _ in range(TOPK):
            m = jnp.max(cv, axis=1, keepdims=True)
            hit = cv == m
            il = jnp.min(jnp.where(hit, ci, IBIG), axis=1, keepdims=True)
            outs.append(il)
            cv = jnp.where(hit & (ci == il), NEG_TAKEN, cv)
        outs += [outs[0]] * (NS - TOPK)
        out_ref[...] = jnp.concatenate(outs, axis=1)


def _topk_call(q, tsk2, train, spk2):
    nchunk = (K + CK - 1) // CK
    ng = nchunk // 2 + 1
    last = nchunk - 1
    return pl.pallas_call(
        _topk_body,
        grid=(ng,),
        in_specs=[
            pl.BlockSpec((B, D), lambda i: (0, 0)),
            pl.BlockSpec((B, 1), lambda i: (0, 0)),
            pl.BlockSpec((CK, D), lambda i: (jnp.minimum(2 * i, last), 0)),
            pl.BlockSpec((1, CK), lambda i: (0, jnp.minimum(2 * i, last))),
            pl.BlockSpec((CK, D), lambda i: (jnp.minimum(2 * i + 1, last), 0)),
            pl.BlockSpec((1, CK), lambda i: (0, jnp.minimum(2 * i + 1, last))),
        ],
        out_specs=pl.BlockSpec((B, NS), lambda i: (0, 0)),
        out_shape=jax.ShapeDtypeStruct((B, NS), jnp.int32),
        scratch_shapes=[
            pltpu.VMEM((B, D), jnp.bfloat16),
            pltpu.VMEM((B, CK), jnp.float32),
            pltpu.VMEM((B, CK), jnp.float32),
            pltpu.VMEM((B, TOPK * LN), jnp.float32),
            pltpu.VMEM((B, TOPK * LN), jnp.float32),
            pltpu.VMEM((B, TOPK * LN), jnp.int32),
        ],
        compiler_params=pltpu.CompilerParams(
            dimension_semantics=("arbitrary",)),
    )(q, tsk2, train, spk2, train, spk2)


_SC_NC = 2
_SC_NSUB = 16
_NW = _SC_NC * _SC_NSUB
_QPW = B // _NW
_RQ = 8
_NROUND = _QPW // _RQ
_ROWS = _RQ * TOPK
_NSL = D // 16


def _gather_mean_body(idx_hbm, tr_hbm, out_hbm, idx_v, rows_v, out_v, sem):
    wid = lax.axis_index("s") * _SC_NC + lax.axis_index("c")
    for r in range(_NROUND):
        base_q = wid * _QPW + r * _RQ
        pltpu.sync_copy(idx_hbm.at[pl.ds(base_q * TOPK, _ROWS)], idx_v)
        pltpu.async_copy(tr_hbm.at[idx_v], rows_v, sem).wait()
        for q in range(_RQ):
            def slice_body(t, carry, q=q):
                sl = pl.ds(t * 16, 16)
                acc = rows_v[TOPK * q, sl]
                for rr in range(1, TOPK):
                    acc = acc + rows_v[TOPK * q + rr, sl]
                out_v[q, sl] = acc * (1.0 / TOPK)
                return carry
            lax.fori_loop(0, _NSL, slice_body, 0)
        pltpu.sync_copy(out_v, out_hbm.at[pl.ds(base_q, _RQ)])


@functools.cache
def _gather_mean_kernel():
    return pl.kernel(
        _gather_mean_body,
        out_type=jax.ShapeDtypeStruct((B, D), jnp.float32),
        mesh=plsc.VectorSubcoreMesh(core_axis_name="c", subcore_axis_name="s",
                                    num_cores=_SC_NC, num_subcores=_SC_NSUB),
        scratch_types=[
            pltpu.VMEM((_ROWS,), jnp.int32),
            pltpu.VMEM((_ROWS, D), jnp.float32),
            pltpu.VMEM((_RQ, D), jnp.float32),
            pltpu.SemaphoreType.DMA,
        ],
    )


def _mlp_body(q_ref, r_ref, w1_ref, b1_ref, w2_ref, b2_ref, out_ref):
    w1 = w1_ref[...]
    h = lax.dot_general(
        q_ref[...], w1[:D, :], (((1,), (0,)), ((), ())),
        precision=lax.Precision.HIGHEST,
        preferred_element_type=jnp.float32)
    h = h + lax.dot_general(
        r_ref[...], w1[D:, :], (((1,), (0,)), ((), ())),
        precision=lax.Precision.HIGHEST,
        preferred_element_type=jnp.float32)
    h = h + b1_ref[...]
    h = h * (1.0 / (1.0 + jnp.exp(-h)))
    out = lax.dot_general(
        h, w2_ref[...], (((1,), (0,)), ((), ())),
        precision=lax.Precision.HIGHEST,
        preferred_element_type=jnp.float32)
    out_ref[...] = out + b2_ref[...]


def _mlp_call(q, retrieved, w1, b1, w2, b2):
    return pl.pallas_call(
        _mlp_body,
        out_shape=jax.ShapeDtypeStruct((B, D), jnp.float32),
    )(q, retrieved, w1, b1.reshape(1, D), w2, b2.reshape(1, D))


def kernel(content_features, target_speaker_id, training_features,
           speaker_ids, W1, b1, W2, b2):
    tsk2 = target_speaker_id.reshape(B, 1)
    spk2 = speaker_ids.reshape(1, K)
    top_idx = _topk_call(content_features, tsk2, training_features, spk2)
    idx_flat = top_idx[:, :TOPK].reshape(B * TOPK)
    retrieved = _gather_mean_kernel()(idx_flat, training_features)
    return _mlp_call(content_features, retrieved, W1, b1, W2, b2)


# speedup vs baseline: 1.1985x; 1.0218x over previous
import functools

import jax
import jax.numpy as jnp
from jax import lax
from jax.experimental import pallas as pl
from jax.experimental.pallas import tpu as pltpu
from jax.experimental.pallas import tpu_sc as plsc

B = 1024
K = 100000
D = 768
TOPK = 5
NS = 8
CK = 1280
EPS = 1e-8
NEG_MASKED = -1.0e30
NEG_TAKEN = -2.0e30
NEG_INIT = -3.0e30
IBIG = 2 ** 30


LN = 128
NCH = CK // LN


def _produce(tr_ref, spk_ref, qn, tsk, bound, dst_ref):
    c = tr_ref[...]
    cnorm2 = jnp.sum(c * c, axis=1)
    rinv = 1.0 / jnp.maximum(jnp.sqrt(cnorm2), EPS)
    cn = (c * rinv[:, None]).astype(jnp.bfloat16)
    dots = lax.dot_general(
        qn, cn, (((1,), (1,)), ((), ())),
        preferred_element_type=jnp.float32)
    inb = lax.broadcasted_iota(jnp.int32, (1, CK), 1) < bound
    spk_adj = jnp.where(inb, spk_ref[...], -1)
    dst_ref[...] = jnp.where(tsk == spk_adj, dots, NEG_MASKED)


HALF = NCH // 2


def _consume(src_ref, ch, lane, V, L, P):
    for k in range(HALF):
        x1 = src_ref[:, k * LN:(k + 1) * LN]
        x2 = src_ref[:, (k + HALF) * LN:(k + HALF + 1) * LN]
        b12 = x2 > x1
        hi = jnp.maximum(x1, x2)
        lo = jnp.minimum(x1, x2)
        pid2 = (lane + (ch * HALF + k) * LN) * 2 + b12.astype(jnp.int32)
        b = [hi > V[t] for t in range(TOPK)]
        nV, nL, nP = [], [], []
        for t in range(TOPK):
            if t == 0:
                pv, pl_, pp = hi, lo, pid2
            else:
                pv = jnp.where(b[t - 1], V[t - 1], hi)
                pl_ = jnp.where(b[t - 1], L[t - 1], lo)
                pp = jnp.where(b[t - 1], P[t - 1], pid2)
            nV.append(jnp.where(b[t], pv, V[t]))
            nL.append(jnp.where(b[t], pl_, L[t]))
            nP.append(jnp.where(b[t], pp, P[t]))
        V, L, P = nV, nL, nP
    return V, L, P


def _topk_body(q_ref, tsk_ref, tr0_ref, spk0_ref, tr1_ref, spk1_ref,
               out_ref, qn_ref, s0_ref, s1_ref, cv_ref, cl_ref, ci_ref):
    i = pl.program_id(0)

    @pl.when(i == 0)
    def _init():
        qv = q_ref[...]
        qnorm = jnp.sqrt(jnp.sum(qv * qv, axis=1, keepdims=True))
        qn_ref[...] = (qv / jnp.maximum(qnorm, EPS)).astype(jnp.bfloat16)
        cv_ref[...] = jnp.full((B, TOPK * LN), NEG_INIT, jnp.float32)
        cl_ref[...] = jnp.full((B, TOPK * LN), NEG_INIT, jnp.float32)
        ci_ref[...] = jnp.zeros((B, TOPK * LN), jnp.int32)
        s1_ref[...] = jnp.full((B, CK), NEG_INIT, jnp.float32)

    qn = qn_ref[...]
    tsk = tsk_ref[...]
    lane = lax.broadcasted_iota(jnp.int32, (1, LN), 1)
    V = [cv_ref[:, t * LN:(t + 1) * LN] for t in range(TOPK)]
    L = [cl_ref[:, t * LN:(t + 1) * LN] for t in range(TOPK)]
    P = [ci_ref[:, t * LN:(t + 1) * LN] for t in range(TOPK)]

    _produce(tr0_ref, spk0_ref, qn, tsk, K - (2 * i) * CK, s0_ref)
    V, L, P = _consume(s1_ref, 2 * i - 1, lane, V, L, P)
    _produce(tr1_ref, spk1_ref, qn, tsk, K - (2 * i + 1) * CK, s1_ref)
    V, L, P = _consume(s0_ref, 2 * i, lane, V, L, P)

    for t in range(TOPK):
        cv_ref[:, t * LN:(t + 1) * LN] = V[t]
        cl_ref[:, t * LN:(t + 1) * LN] = L[t]
        ci_ref[:, t * LN:(t + 1) * LN] = P[t]

    @pl.when(i == pl.num_programs(0) - 1)
    def _emit():
        pid2 = ci_ref[...]
        flag = pid2 & 1
        pid = pid2 >> 1
        g = pid >> 7
        ln_ = pid & (LN - 1)
        c1 = (g // HALF) * CK + (g % HALF) * LN + ln_
        hic = c1 + flag * (HALF * LN)
        loc = c1 + (1 - flag) * (HALF * LN)
        cv = jnp.concatenate([cv_ref[...], cl_ref[...]], axis=1)
        ci = jnp.concatenate([hic, loc], axis=1)
        outs = []
        for _ in range(TOPK):
            m = jnp.max(cv, axis=1, keepdims=True)
            hit = cv == m
            il = jnp.min(jnp.where(hit, ci, IBIG), axis=1, keepdims=True)
            outs.append(il)
            cv = jnp.where(hit & (ci == il), NEG_TAKEN, cv)
        outs += [outs[0]] * (NS - TOPK)
        out_ref[...] = jnp.concatenate(outs, axis=1)


def _topk_call(q, tsk2, train, spk2):
    nchunk = (K + CK - 1) // CK
    ng = nchunk // 2 + 1
    last = nchunk - 1
    return pl.pallas_call(
        _topk_body,
        grid=(ng,),
        in_specs=[
            pl.BlockSpec((B, D), lambda i: (0, 0)),
            pl.BlockSpec((B, 1), lambda i: (0, 0)),
            pl.BlockSpec((CK, D), lambda i: (jnp.minimum(2 * i, last), 0)),
            pl.BlockSpec((1, CK), lambda i: (0, jnp.minimum(2 * i, last))),
            pl.BlockSpec((CK, D), lambda i: (jnp.minimum(2 * i + 1, last), 0)),
            pl.BlockSpec((1, CK), lambda i: (0, jnp.minimum(2 * i + 1, last))),
        ],
        out_specs=pl.BlockSpec((B, NS), lambda i: (0, 0)),
        out_shape=jax.ShapeDtypeStruct((B, NS), jnp.int32),
        scratch_shapes=[
            pltpu.VMEM((B, D), jnp.bfloat16),
            pltpu.VMEM((B, CK), jnp.float32),
            pltpu.VMEM((B, CK), jnp.float32),
            pltpu.VMEM((B, TOPK * LN), jnp.float32),
            pltpu.VMEM((B, TOPK * LN), jnp.float32),
            pltpu.VMEM((B, TOPK * LN), jnp.int32),
        ],
        compiler_params=pltpu.CompilerParams(
            dimension_semantics=("arbitrary",)),
    )(q, tsk2, train, spk2, train, spk2)


_SC_NC = 2
_SC_NSUB = 16
_NW = _SC_NC * _SC_NSUB
_QPW = B // _NW
_RQ = 8
_NROUND = _QPW // _RQ
_ROWS = _RQ * TOPK
_NSL = D // 16


def _gather_mean_body(idx_hbm, tr_hbm, out_hbm, idx_v, rows_v, out_v, sem):
    wid = lax.axis_index("s") * _SC_NC + lax.axis_index("c")
    for r in range(_NROUND):
        base_q = wid * _QPW + r * _RQ
        pltpu.sync_copy(idx_hbm.at[pl.ds(base_q * TOPK, _ROWS)], idx_v)
        pltpu.async_copy(tr_hbm.at[idx_v], rows_v, sem).wait()
        for q in range(_RQ):
            def slice_body(t, carry, q=q):
                sl = pl.ds(t * 16, 16)
                acc = rows_v[TOPK * q, sl]
                for rr in range(1, TOPK):
                    acc = acc + rows_v[TOPK * q + rr, sl]
                out_v[q, sl] = acc * (1.0 / TOPK)
                return carry
            lax.fori_loop(0, _NSL, slice_body, 0)
        pltpu.sync_copy(out_v, out_hbm.at[pl.ds(base_q, _RQ)])


@functools.cache
def _gather_mean_kernel():
    return pl.kernel(
        _gather_mean_body,
        out_type=jax.ShapeDtypeStruct((B, D), jnp.float32),
        mesh=plsc.VectorSubcoreMesh(core_axis_name="c", subcore_axis_name="s",
                                    num_cores=_SC_NC, num_subcores=_SC_NSUB),
        scratch_types=[
            pltpu.VMEM((_ROWS,), jnp.int32),
            pltpu.VMEM((_ROWS, D), jnp.float32),
            pltpu.VMEM((_RQ, D), jnp.float32),
            pltpu.SemaphoreType.DMA,
        ],
    )


def _mlp_a_body(q_ref, w1a_ref, b1_ref, h_ref):
    h = lax.dot_general(
        q_ref[...], w1a_ref[...], (((1,), (0,)), ((), ())),
        precision=lax.Precision.HIGHEST,
        preferred_element_type=jnp.float32)
    h_ref[...] = h + b1_ref[...]


def _mlp_b_body(h1_ref, r_ref, w1b_ref, w2_ref, b2_ref, out_ref):
    h = h1_ref[...] + lax.dot_general(
        r_ref[...], w1b_ref[...], (((1,), (0,)), ((), ())),
        precision=lax.Precision.HIGHEST,
        preferred_element_type=jnp.float32)
    h = h * (1.0 / (1.0 + jnp.exp(-h)))
    out = lax.dot_general(
        h, w2_ref[...], (((1,), (0,)), ((), ())),
        precision=lax.Precision.HIGHEST,
        preferred_element_type=jnp.float32)
    out_ref[...] = out + b2_ref[...]


def _mlp_a_call(q, w1, b1):
    return pl.pallas_call(
        _mlp_a_body,
        out_shape=jax.ShapeDtypeStruct((B, D), jnp.float32),
    )(q, w1[:D], b1.reshape(1, D))


def _mlp_b_call(h1, retrieved, w1, w2, b2):
    return pl.pallas_call(
        _mlp_b_body,
        out_shape=jax.ShapeDtypeStruct((B, D), jnp.float32),
    )(h1, retrieved, w1[D:], w2, b2.reshape(1, D))


def kernel(content_features, target_speaker_id, training_features,
           speaker_ids, W1, b1, W2, b2):
    tsk2 = target_speaker_id.reshape(B, 1)
    spk2 = speaker_ids.reshape(1, K)
    top_idx = _topk_call(content_features, tsk2, training_features, spk2)
    idx_flat = top_idx[:, :TOPK].reshape(B * TOPK)
    retrieved = _gather_mean_kernel()(idx_flat, training_features)
    h1 = _mlp_a_call(content_features, W1, b1)
    return _mlp_b_call(h1, retrieved, W1, W2, b2)
